# baseline (speedup 1.0000x reference)
import jax
import jax.numpy as jnp
from jax import lax
from jax.experimental import pallas as pl
from jax.experimental.pallas import tpu as pltpu
from jax.experimental.pallas import tpu_sc as plsc

N = 10000
D = 256
DH = 128
E = 160000
NC = 2
NS = 16
L = 16
CHUNK = 80
CPT = 125
EPT = CPT * CHUNK
CHUNKC = 40
CPTC = 125
ROWS = 10112
RPT = ROWS // NS
ROWS_C = 10240
RPT_C = ROWS_C // NS
BT = 1000


def _zero_fill(zb, width):
    zero16 = jnp.zeros((L,), jnp.float32)
    for r in range(16):
        for k in range(width // L):
            zb[r, pl.ds(k * L, L)] = zero16


def _zero_acc(zb, acc, r0):
    for k in range(RPT // 16):
        pltpu.sync_copy(zb, acc.at[pl.ds(r0 + k * 16, 16)])
    rem = RPT % 16
    if rem:
        pltpu.sync_copy(zb.at[pl.ds(0, rem)], acc.at[pl.ds(r0 + RPT - rem, rem)])


def _fused_body(xs_u, xs_v, src_a, dst_a, src_d, dst_d,
                sum_a, cnt_a, sum_d, cnt_d,
                sidx, didx, rows, zb, acc, sem):
    c = lax.axis_index("c")
    s = lax.axis_index("s")
    _zero_fill(zb, DH)
    r0 = s * RPT
    cb = jnp.full((L,), c, jnp.int32)
    one16 = jnp.ones((L,), jnp.float32)

    def stage(src_hbm, dst_hbm):
        pltpu.sync_copy(src_hbm.at[s], sidx)
        pltpu.sync_copy(dst_hbm.at[s], didx)

        def fix(t, carry):
            v = sidx[pl.ds(t * L, L)]
            sidx[pl.ds(t * L, L)] = v + cb
            return carry

        lax.fori_loop(0, EPT // L, fix, None)

    def sum_loop(xs_hbm):
        def gref(a):
            return xs_hbm.at[sidx.at[pl.ds(a * CHUNK, CHUNK)]]

        pltpu.async_copy(gref(0), rows.at[0], sem.at[0])

        def chunk(j, carry):
            b = lax.rem(j, 2)
            bn = lax.rem(j + 1, 2)

            @pl.when(j < CPT - 1)
            def _():
                pltpu.async_copy(gref(j + 1), rows.at[bn], sem.at[bn])
            pltpu.make_async_copy(gref(j), rows.at[b], sem.at[b]).wait()
            pltpu.sync_copy(rows.at[b], acc.at[didx.at[j]], add=True)
            return carry

        lax.fori_loop(0, CPT, chunk, None)

    def cnt_loop():
        for r in range(CHUNK):
            rows[1, r, pl.ds(0, L)] = one16
        lo = c * (CPT // 2 + 1)
        hi = jnp.minimum(lo + CPT // 2 + 1, CPT)

        def chunk(j, carry):
            pltpu.sync_copy(rows.at[1], acc.at[didx.at[j]], add=True)
            return carry

        lax.fori_loop(lo, hi, chunk, None)

    def writeout(out_hbm):
        pltpu.sync_copy(acc.at[pl.ds(r0, RPT)], out_hbm.at[c, pl.ds(r0, RPT)])

    for (src_r, dst_r, xs_r, sum_r, cnt_r) in (
            (src_a, dst_a, xs_u, sum_a, cnt_a),
            (src_d, dst_d, xs_v, sum_d, cnt_d)):
        stage(src_r, dst_r)
        _zero_acc(zb, acc, r0)
        plsc.subcore_barrier()
        sum_loop(xs_r)
        plsc.subcore_barrier()
        writeout(sum_r)
        _zero_acc(zb, acc, r0)
        plsc.subcore_barrier()
        cnt_loop()
        plsc.subcore_barrier()
        writeout(cnt_r)


_sc_fused = pl.kernel(
    _fused_body,
    out_type=(
        jax.ShapeDtypeStruct((NC, ROWS, DH), jnp.float32),
        jax.ShapeDtypeStruct((NC, ROWS, DH), jnp.float32),
        jax.ShapeDtypeStruct((NC, ROWS, DH), jnp.float32),
        jax.ShapeDtypeStruct((NC, ROWS, DH), jnp.float32),
    ),
    mesh=plsc.VectorSubcoreMesh(core_axis_name="c", subcore_axis_name="s"),
    scratch_types=[
        pltpu.VMEM((EPT,), jnp.int32),
        pltpu.VMEM((CPT, CHUNK), jnp.int32),
        pltpu.VMEM((2, CHUNK, DH), jnp.float32),
        pltpu.VMEM((16, DH), jnp.float32),
        pltpu.VMEM_SHARED((ROWS, DH), jnp.float32),
        pltpu.SemaphoreType.DMA((2,)),
    ],
)


def _tc_body(slo_ref, shi_ref, c0_ref, c1_ref, x_ref, wlo_ref, whi_ref,
             wr_ref, bl_ref, g_ref, b_ref, o_ref):
    t = jnp.dot(slo_ref[0], wlo_ref[...], preferred_element_type=jnp.float32)
    t = t + jnp.dot(shi_ref[0], whi_ref[...], preferred_element_type=jnp.float32)
    cnt = (c0_ref[0][:, 0:1] + c1_ref[0][:, 0:1]).astype(jnp.float32)
    rec = 1.0 / jnp.maximum(cnt, 1.0)
    h = (t * rec + bl_ref[...]
         + jnp.dot(x_ref[...], wr_ref[...], preferred_element_type=jnp.float32))
    mu = jnp.mean(h, axis=-1, keepdims=True)
    d = h - mu
    var = jnp.mean(d * d, axis=-1, keepdims=True)
    y = d * lax.rsqrt(var + 1e-5) * g_ref[...] + b_ref[...]
    o_ref[...] = jnp.maximum(y, 0.0)


_encode_tc = pl.pallas_call(
    _tc_body,
    grid=(N // BT,),
    in_specs=[
        pl.BlockSpec((1, BT, DH), lambda i: (0, i, 0)),
        pl.BlockSpec((1, BT, DH), lambda i: (1, i, 0)),
        pl.BlockSpec((1, BT, DH), lambda i: (0, i, 0)),
        pl.BlockSpec((1, BT, DH), lambda i: (1, i, 0)),
        pl.BlockSpec((BT, D), lambda i: (i, 0)),
        pl.BlockSpec((DH, D), lambda i: (0, 0)),
        pl.BlockSpec((DH, D), lambda i: (0, 0)),
        pl.BlockSpec((D, D), lambda i: (0, 0)),
        pl.BlockSpec((1, D), lambda i: (0, 0)),
        pl.BlockSpec((1, D), lambda i: (0, 0)),
        pl.BlockSpec((1, D), lambda i: (0, 0)),
    ],
    out_specs=pl.BlockSpec((BT, D), lambda i: (i, 0)),
    out_shape=jax.ShapeDtypeStruct((N, D), jnp.float32),
)


def _prep_edges(edge_index):
    src = edge_index[0].astype(jnp.int32)
    dst = edge_index[1].astype(jnp.int32)
    srcp = (2 * src).reshape(NS, EPT)
    dstp = dst.reshape(NS, CPT, CHUNK)
    return srcp, dstp


def _encode(sum3, cnt3, x_dst, Wl, bl, Wr, gamma, beta):
    return _encode_tc(
        sum3, sum3, cnt3, cnt3, x_dst,
        Wl[:, :DH].T, Wl[:, DH:].T, Wr.T,
        bl.reshape(1, D), gamma.reshape(1, D), beta.reshape(1, D))


def kernel(x_u, x_v, edge_index_adv, edge_index_dif,
           Wl_adv, bl_adv, Wr_adv, Wl_dif, bl_dif, Wr_dif,
           gamma, beta):
    src_a, dst_a = _prep_edges(edge_index_adv)
    src_d, dst_d = _prep_edges(edge_index_dif)
    sum_a, cnt_a, sum_d, cnt_d = _sc_fused(
        x_u.reshape(2 * N, DH), x_v.reshape(2 * N, DH),
        src_a, dst_a, src_d, dst_d)
    h_adv = _encode(sum_a, cnt_a, x_v, Wl_adv, bl_adv, Wr_adv, gamma, beta)
    h_dif = _encode(sum_d, cnt_d, x_u, Wl_dif, bl_dif, Wr_dif, gamma, beta)
    return (h_adv, h_dif)
